# baseline (speedup 1.0000x reference)
from functools import partial

import jax
import jax.numpy as jnp
from jax import lax
from jax.experimental import pallas as pl
from jax.experimental.pallas import tpu as pltpu

_EPS = 1e-5


def _chan_stats(y):
    return jnp.concatenate([jnp.sum(y, axis=1, keepdims=True),
                            jnp.sum(y * y, axis=1, keepdims=True)], axis=1)


def _fold(stats, gamma_beta, count):
    mu = stats[:, 0:1] / count
    var = jnp.maximum(stats[:, 1:2] / count - mu * mu, 0.0)
    scale = gamma_beta[:, 0:1] * lax.rsqrt(var + _EPS)
    shift = gamma_beta[:, 1:2] - mu * scale
    return jnp.concatenate([scale, shift], axis=1)


def _fused_kernel(x_ref, w13l_ref, w13ce_ref, w2l_ref, w2c_ref, w2r_ref,
                  bn1_ref, bn2_ref, o_ref,
                  y13_scr, y2_scr, st1_scr, st2_scr, ss1_scr, ss2_scr,
                  *, TN, C_in, C_half, C_out, L_out, count):
    p = pl.program_id(0)
    i = pl.program_id(1)

    @pl.when(jnp.logical_and(p == 0, i == 0))
    def _():
        st1_scr[...] = jnp.zeros_like(st1_scr)
        st2_scr[...] = jnp.zeros_like(st2_scr)

    @pl.when(p == 0)
    def _():
        w_l = w13l_ref[...]
        w_ce = w13ce_ref[...]
        lane = lax.broadcasted_iota(jnp.int32, (C_in, L_out), 1)
        not_first = lane > 0
        acc = jnp.zeros((C_half, 2), jnp.float32)
        for s in range(TN):
            xi = x_ref[s]
            xil = jnp.where(not_first, pltpu.roll(xi, shift=1, axis=1),
                            jnp.uint32(0))
            xb = pltpu.bitcast(xi, jnp.bfloat16)
            xlb = pltpu.bitcast(xil, jnp.bfloat16)
            y = (jnp.dot(w_ce, xb, preferred_element_type=jnp.float32)
                 + jnp.dot(w_l, xlb, preferred_element_type=jnp.float32))
            acc = acc + _chan_stats(y[:C_half, :])
            y13_scr[i * TN + s] = y.astype(jnp.bfloat16)
        st1_scr[...] += acc

    @pl.when(jnp.logical_and(p == 1, i == 0))
    def _():
        ss1_scr[...] = _fold(st1_scr[...], bn1_ref[...], count)

    @pl.when(p == 1)
    def _():
        ss1 = ss1_scr[...]
        scale1, shift1 = ss1[:, 0:1], ss1[:, 1:2]
        w2l = w2l_ref[...]
        w2c = w2c_ref[...]
        w2r = w2r_ref[...]
        lane = lax.broadcasted_iota(jnp.int32, (C_half, L_out), 1)
        not_first = lane > 0
        not_last = lane < L_out - 1
        acc = jnp.zeros((C_out, 2), jnp.float32)
        for s in range(TN):
            y13 = y13_scr[i * TN + s]
            a = y13[:C_half, :].astype(jnp.float32) * scale1 + shift1
            h = (a * jax.nn.sigmoid(a)).astype(jnp.bfloat16)
            h_l = jnp.where(not_first, pltpu.roll(h, shift=1, axis=1),
                            jnp.bfloat16(0))
            h_r = jnp.where(not_last, pltpu.roll(h, shift=L_out - 1, axis=1),
                            jnp.bfloat16(0))
            y2 = (jnp.dot(w2c, h, preferred_element_type=jnp.float32)
                  + jnp.dot(w2l, h_l, preferred_element_type=jnp.float32)
                  + jnp.dot(w2r, h_r, preferred_element_type=jnp.float32)
                  + y13[C_half:, :].astype(jnp.float32))
            acc = acc + _chan_stats(y2)
            y2_scr[i * TN + s] = y2.astype(jnp.bfloat16)
        st2_scr[...] += acc

    @pl.when(jnp.logical_and(p == 2, i == 0))
    def _():
        ss2_scr[...] = _fold(st2_scr[...], bn2_ref[...], count)

    @pl.when(p == 2)
    def _():
        ss2 = ss2_scr[...]
        scale2, shift2 = ss2[:, 0:1], ss2[:, 1:2]
        for s in range(TN):
            a = y2_scr[i * TN + s].astype(jnp.float32) * scale2 + shift2
            o_ref[s] = a * jax.nn.sigmoid(a)


def kernel(x, w1, b1, g1, be1, w2, b2, g2, be2, w3, b3):
    N, C_in, L = x.shape
    C_half = w1.shape[0]
    C_out = w2.shape[0]
    L_out = (L + 1) // 2
    C13 = C_half + C_out
    count = float(N * L_out)

    xb = x.astype(jnp.bfloat16)
    if L % 2:
        xb = jnp.pad(xb, ((0, 0), (0, 0), (0, 1)))
    xp = lax.bitcast_convert_type(xb.reshape(N, C_in, L_out, 2), jnp.uint32)

    w13 = jnp.concatenate([w1, w3], axis=0)
    w13l = jnp.stack([jnp.zeros_like(w13[:, :, 0]), w13[:, :, 0]],
                     axis=2).reshape(C13, 2 * C_in).astype(jnp.bfloat16)
    w13ce = jnp.stack([w13[:, :, 1], w13[:, :, 2]],
                      axis=2).reshape(C13, 2 * C_in).astype(jnp.bfloat16)
    w2l = w2[:, :, 0].astype(jnp.bfloat16)
    w2c = w2[:, :, 1].astype(jnp.bfloat16)
    w2r = w2[:, :, 2].astype(jnp.bfloat16)
    bn1p = jnp.stack([g1, be1], axis=1).astype(jnp.float32)
    bn2p = jnp.stack([g2, be2], axis=1).astype(jnp.float32)

    TN = 1
    for d in range(1, min(N, 8) + 1):
        if N % d == 0:
            TN = d
    n_tiles = N // TN

    return pl.pallas_call(
        partial(_fused_kernel, TN=TN, C_in=C_in, C_half=C_half, C_out=C_out,
                L_out=L_out, count=count),
        grid=(3, n_tiles),
        in_specs=[
            pl.BlockSpec((TN, C_in, L_out),
                         lambda p, i: (i * ((2 - p) // 2), 0, 0)),
            pl.BlockSpec((C13, 2 * C_in), lambda p, i: (0, 0)),
            pl.BlockSpec((C13, 2 * C_in), lambda p, i: (0, 0)),
            pl.BlockSpec((C_out, C_half), lambda p, i: (0, 0)),
            pl.BlockSpec((C_out, C_half), lambda p, i: (0, 0)),
            pl.BlockSpec((C_out, C_half), lambda p, i: (0, 0)),
            pl.BlockSpec((C_half, 2), lambda p, i: (0, 0)),
            pl.BlockSpec((C_out, 2), lambda p, i: (0, 0)),
        ],
        out_specs=pl.BlockSpec((TN, C_out, L_out),
                               lambda p, i: (i * (p // 2), 0, 0)),
        out_shape=jax.ShapeDtypeStruct((N, C_out, L_out), jnp.float32),
        scratch_shapes=[
            pltpu.VMEM((N, C13, L_out), jnp.bfloat16),
            pltpu.VMEM((N, C_out, L_out), jnp.bfloat16),
            pltpu.VMEM((C_half, 2), jnp.float32),
            pltpu.VMEM((C_out, 2), jnp.float32),
            pltpu.VMEM((C_half, 2), jnp.float32),
            pltpu.VMEM((C_out, 2), jnp.float32),
        ],
        compiler_params=pltpu.CompilerParams(
            dimension_semantics=("arbitrary", "arbitrary"),
            vmem_limit_bytes=64 * 2**20),
    )(xp, w13l, w13ce, w2l, w2c, w2r, bn1p, bn2p)


# speedup vs baseline: 1.3620x; 1.3620x over previous
from functools import partial

import jax
import jax.numpy as jnp
from jax import lax
from jax.experimental import pallas as pl
from jax.experimental.pallas import tpu as pltpu

_EPS = 1e-5


def _chan_stats(y):
    return jnp.concatenate([jnp.sum(y, axis=1, keepdims=True),
                            jnp.sum(y * y, axis=1, keepdims=True)], axis=1)


def _fold(stats, gamma_beta, count):
    mu = stats[:, 0:1] / count
    var = jnp.maximum(stats[:, 1:2] / count - mu * mu, 0.0)
    scale = gamma_beta[:, 0:1] * lax.rsqrt(var + _EPS)
    shift = gamma_beta[:, 1:2] - mu * scale
    return jnp.concatenate([scale, shift], axis=1)


def _fused_kernel(x_ref, w13l_ref, w13ce_ref, w2l_ref, w2c_ref, w2r_ref,
                  bn1_ref, bn2_ref, o_ref,
                  ybuf, st1_scr, st2_scr, ss1_scr, ss2_scr,
                  *, TN, C_in, C_half, C_out, L_out, count):
    p = pl.program_id(0)
    i = pl.program_id(1)

    @pl.when(jnp.logical_and(p == 0, i == 0))
    def _():
        st1_scr[...] = jnp.zeros_like(st1_scr)
        st2_scr[...] = jnp.zeros_like(st2_scr)

    @pl.when(p == 0)
    def _():
        w_l = w13l_ref[...]
        w_ce = w13ce_ref[...]
        lane = lax.broadcasted_iota(jnp.int32, (C_in, L_out), 1)
        not_first = lane > 0
        acc = jnp.zeros((C_half, 2), jnp.float32)
        for s in range(TN):
            x_eo = x_ref[s]
            xo = x_eo[C_in:, :]
            xl = jnp.where(not_first, pltpu.roll(xo, shift=1, axis=1),
                           jnp.bfloat16(0))
            y = (jnp.dot(w_ce, x_eo, preferred_element_type=jnp.float32)
                 + jnp.dot(w_l, xl, preferred_element_type=jnp.float32))
            acc = acc + _chan_stats(y[:C_half, :])
            ybuf[i * TN + s] = y.astype(jnp.bfloat16)
        st1_scr[...] += acc

    @pl.when(jnp.logical_and(p == 1, i == 0))
    def _():
        ss1_scr[...] = _fold(st1_scr[...], bn1_ref[...], count)

    @pl.when(p == 1)
    def _():
        ss1 = ss1_scr[...]
        scale1, shift1 = ss1[:, 0:1], ss1[:, 1:2]
        w2l = w2l_ref[...]
        w2c = w2c_ref[...]
        w2r = w2r_ref[...]
        lane = lax.broadcasted_iota(jnp.int32, (C_half, L_out), 1)
        not_first = lane > 0
        not_last = lane < L_out - 1
        acc = jnp.zeros((C_out, 2), jnp.float32)
        for s in range(TN):
            y13 = ybuf[i * TN + s]
            a = y13[:C_half, :].astype(jnp.float32) * scale1 + shift1
            h = (a * jax.nn.sigmoid(a)).astype(jnp.bfloat16)
            h_l = jnp.where(not_first, pltpu.roll(h, shift=1, axis=1),
                            jnp.bfloat16(0))
            h_r = jnp.where(not_last, pltpu.roll(h, shift=L_out - 1, axis=1),
                            jnp.bfloat16(0))
            y2 = (jnp.dot(w2c, h, preferred_element_type=jnp.float32)
                  + jnp.dot(w2l, h_l, preferred_element_type=jnp.float32)
                  + jnp.dot(w2r, h_r, preferred_element_type=jnp.float32)
                  + y13[C_half:, :].astype(jnp.float32))
            acc = acc + _chan_stats(y2)
            ybuf[i * TN + s, :C_out, :] = y2.astype(jnp.bfloat16)
        st2_scr[...] += acc

    @pl.when(jnp.logical_and(p == 2, i == 0))
    def _():
        ss2_scr[...] = _fold(st2_scr[...], bn2_ref[...], count)

    @pl.when(p == 2)
    def _():
        ss2 = ss2_scr[...]
        scale2, shift2 = ss2[:, 0:1], ss2[:, 1:2]
        for s in range(TN):
            y2 = ybuf[i * TN + s, :C_out, :]
            a = y2.astype(jnp.float32) * scale2 + shift2
            o_ref[s] = a * jax.nn.sigmoid(a)


def kernel(x, w1, b1, g1, be1, w2, b2, g2, be2, w3, b3):
    N, C_in, L = x.shape
    C_half = w1.shape[0]
    C_out = w2.shape[0]
    L_out = (L + 1) // 2
    C13 = C_half + C_out
    count = float(N * L_out)

    xb = x.astype(jnp.bfloat16)
    if L % 2:
        xb = jnp.pad(xb, ((0, 0), (0, 0), (0, 1)))
    x_eo = (xb.reshape(N, C_in, L_out, 2)
              .transpose(0, 3, 1, 2)
              .reshape(N, 2 * C_in, L_out))

    w13 = jnp.concatenate([w1, w3], axis=0)
    w13l = w13[:, :, 0].astype(jnp.bfloat16)
    w13ce = jnp.concatenate([w13[:, :, 1], w13[:, :, 2]],
                            axis=1).astype(jnp.bfloat16)
    w2l = w2[:, :, 0].astype(jnp.bfloat16)
    w2c = w2[:, :, 1].astype(jnp.bfloat16)
    w2r = w2[:, :, 2].astype(jnp.bfloat16)
    bn1p = jnp.stack([g1, be1], axis=1).astype(jnp.float32)
    bn2p = jnp.stack([g2, be2], axis=1).astype(jnp.float32)

    TN = 1
    for d in range(1, min(N, 16) + 1):
        if N % d == 0:
            TN = d
    n_tiles = N // TN

    return pl.pallas_call(
        partial(_fused_kernel, TN=TN, C_in=C_in, C_half=C_half, C_out=C_out,
                L_out=L_out, count=count),
        grid=(3, n_tiles),
        in_specs=[
            pl.BlockSpec((TN, 2 * C_in, L_out),
                         lambda p, i: (i * ((2 - p) // 2), 0, 0)),
            pl.BlockSpec((C13, C_in), lambda p, i: (0, 0)),
            pl.BlockSpec((C13, 2 * C_in), lambda p, i: (0, 0)),
            pl.BlockSpec((C_out, C_half), lambda p, i: (0, 0)),
            pl.BlockSpec((C_out, C_half), lambda p, i: (0, 0)),
            pl.BlockSpec((C_out, C_half), lambda p, i: (0, 0)),
            pl.BlockSpec((C_half, 2), lambda p, i: (0, 0)),
            pl.BlockSpec((C_out, 2), lambda p, i: (0, 0)),
        ],
        out_specs=pl.BlockSpec((TN, C_out, L_out),
                               lambda p, i: (i * (p // 2), 0, 0)),
        out_shape=jax.ShapeDtypeStruct((N, C_out, L_out), jnp.float32),
        scratch_shapes=[
            pltpu.VMEM((N, C13, L_out), jnp.bfloat16),
            pltpu.VMEM((C_half, 2), jnp.float32),
            pltpu.VMEM((C_out, 2), jnp.float32),
            pltpu.VMEM((C_half, 2), jnp.float32),
            pltpu.VMEM((C_out, 2), jnp.float32),
        ],
        compiler_params=pltpu.CompilerParams(
            dimension_semantics=("arbitrary", "arbitrary"),
            vmem_limit_bytes=64 * 2**20),
    )(x_eo, w13l, w13ce, w2l, w2c, w2r, bn1p, bn2p)
